# baseline (speedup 1.0000x reference)
import jax
import jax.numpy as jnp
from jax import lax
from jax.experimental import pallas as pl
from jax.experimental.pallas import tpu as pltpu
from jax.experimental.pallas import tpu_sc as plsc

DIM = 200
SEQ = 16906
N = SEQ * DIM
NW = 32
CHUNK = 105656
TAIL = N - NW * CHUNK


def _copy_body(src_hbm, out_hbm, buf):
    wid = lax.axis_index("s") * 2 + lax.axis_index("c")
    base = wid * CHUNK
    pltpu.sync_copy(src_hbm.at[pl.ds(base, CHUNK)], buf)
    pltpu.sync_copy(buf, out_hbm.at[pl.ds(base, CHUNK)])

    @pl.when(wid == 0)
    def _tail():
        tbuf = buf.at[pl.ds(0, TAIL)]
        pltpu.sync_copy(src_hbm.at[pl.ds(NW * CHUNK, TAIL)], tbuf)
        pltpu.sync_copy(tbuf, out_hbm.at[pl.ds(NW * CHUNK, TAIL)])


def kernel(x, table):
    del x
    flat = table.reshape(-1)
    run = pl.kernel(
        _copy_body,
        out_type=jax.ShapeDtypeStruct((N,), jnp.float32),
        mesh=plsc.VectorSubcoreMesh(core_axis_name="c", subcore_axis_name="s"),
        scratch_types=[pltpu.VMEM((CHUNK,), jnp.float32)],
    )
    return run(flat).reshape(SEQ, DIM)


# speedup vs baseline: 3.7999x; 3.7999x over previous
import jax
import jax.numpy as jnp
from jax import lax
from jax.experimental import pallas as pl
from jax.experimental.pallas import tpu as pltpu
from jax.experimental.pallas import tpu_sc as plsc

DIM = 200
SEQ = 16906
NW = 32
ROWS = 264
NCHUNK = 2
EXTRA_OFF = NW * NCHUNK * ROWS
ALIGNED = EXTRA_OFF + 8
TC_BLK = 8


def _sc_body(src_hbm, out_hbm, buf, tbuf):
    wid = lax.axis_index("s") * 2 + lax.axis_index("c")
    for k in range(NCHUNK):
        base = (wid * NCHUNK + k) * ROWS
        pltpu.sync_copy(src_hbm.at[pl.ds(base, ROWS)], buf)
        pltpu.sync_copy(buf, out_hbm.at[pl.ds(base, ROWS)])

    @pl.when(wid == 0)
    def _extra():
        pltpu.sync_copy(src_hbm.at[pl.ds(EXTRA_OFF, 8)], tbuf)
        pltpu.sync_copy(tbuf, out_hbm.at[pl.ds(EXTRA_OFF, 8)])


def _tc_tail_body(part_ref, src_ref, out_ref):
    del part_ref
    out_ref[...] = src_ref[...]


def kernel(x, table):
    del x
    sc_run = pl.kernel(
        _sc_body,
        out_type=jax.ShapeDtypeStruct((SEQ, DIM), jnp.float32),
        mesh=plsc.VectorSubcoreMesh(core_axis_name="c", subcore_axis_name="s"),
        scratch_types=[pltpu.VMEM((ROWS, DIM), jnp.float32),
                       pltpu.VMEM((8, DIM), jnp.float32)],
    )
    part = sc_run(table)
    blk_idx = ALIGNED // TC_BLK
    return pl.pallas_call(
        _tc_tail_body,
        grid=(1,),
        in_specs=[
            pl.BlockSpec((TC_BLK, DIM), lambda i: (blk_idx, 0)),
            pl.BlockSpec((TC_BLK, DIM), lambda i: (blk_idx, 0)),
        ],
        out_specs=pl.BlockSpec((TC_BLK, DIM), lambda i: (blk_idx, 0)),
        out_shape=jax.ShapeDtypeStruct((SEQ, DIM), jnp.float32),
        input_output_aliases={0: 0},
    )(part, table)
